# baseline (speedup 1.0000x reference)
import jax
import jax.numpy as jnp
from jax import lax
from jax.experimental import pallas as pl
from jax.experimental.pallas import tpu as pltpu
from jax.experimental.pallas import tpu_sc as plsc

B, T, D = 4, 8192, 1024
NC, NS = 2, 16
NW = NC * NS
TPW = T // NW
TC_ROWS = 32
HALF = TC_ROWS // 2
NCHUNK = TPW // TC_ROWS
VREGS_H = HALF * D // 16
NSTEP = NCHUNK * B


def _sc_body(x_hbm, pe_hbm, out_hbm,
             xb0, xb1, pb0, pb1,
             sl0, sl1, ss0a, ss0b, ss1a, ss1b, sp0, sp1):
    xbufs, pbufs = (xb0, xb1), (pb0, pb1)
    lsems = (sl0, sl1)
    ssems = ((ss0a, ss0b), (ss1a, ss1b))
    psems = (sp0, sp1)
    wid = lax.axis_index("s") * NC + lax.axis_index("c")
    t0 = wid * TPW

    def rows(s):
        tc, b = divmod(s, B)
        return b, t0 + tc * TC_ROWS

    def start_load(s):
        b, r = rows(s)
        return pltpu.async_copy(x_hbm.at[b, pl.ds(r, TC_ROWS)],
                                xbufs[s % 2], lsems[s % 2])

    def start_store_half(s, h):
        b, r = rows(s)
        return pltpu.async_copy(
            xbufs[s % 2].at[pl.ds(h * HALF, HALF)],
            out_hbm.at[b, pl.ds(r + h * HALF, HALF)],
            ssems[s % 2][h])

    def start_pe(tc):
        return pltpu.async_copy(
            pe_hbm.at[pl.ds(t0 + tc * TC_ROWS, TC_ROWS)],
            pbufs[tc % 2], psems[tc % 2])

    def add_half(x_buf, pe_buf, h):
        @plsc.parallel_loop(0, VREGS_H, step=1, unroll=16)
        def _add(i):
            r = h * HALF + (i >> 6)
            c = (i & 63) * 16
            plsc.addupdate(x_buf.at[r, pl.ds(c, 16)],
                           pe_buf[r, pl.ds(c, 16)])

    pe_handles = {0: start_pe(0)}
    load_handles = {0: start_load(0)}
    store_handles = {}
    for s in range(NSTEP):
        tc, b = divmod(s, B)
        if b == 0:
            pe_handles[tc].wait()
            if tc + 1 < NCHUNK:
                pe_handles[tc + 1] = start_pe(tc + 1)
        if s + 1 < NSTEP:
            if s >= 1:
                store_handles[s - 1][0].wait()
                store_handles[s - 1][1].wait()
            load_handles[s + 1] = start_load(s + 1)
        load_handles[s].wait()

        x_buf, pe_buf = xbufs[s % 2], pbufs[tc % 2]
        add_half(x_buf, pe_buf, 0)
        h0 = start_store_half(s, 0)
        add_half(x_buf, pe_buf, 1)
        h1 = start_store_half(s, 1)
        store_handles[s] = (h0, h1)
    store_handles[NSTEP - 2][0].wait()
    store_handles[NSTEP - 2][1].wait()
    store_handles[NSTEP - 1][0].wait()
    store_handles[NSTEP - 1][1].wait()


def kernel(x, pe_table):
    mesh = plsc.VectorSubcoreMesh(
        core_axis_name="c", subcore_axis_name="s",
        num_cores=NC, num_subcores=NS)
    buf = pltpu.VMEM((TC_ROWS, D), jnp.float32)
    return pl.kernel(
        _sc_body,
        out_type=jax.ShapeDtypeStruct((B, T, D), jnp.float32),
        mesh=mesh,
        scratch_types=[buf] * 4 + [pltpu.SemaphoreType.DMA] * 8,
    )(x, pe_table)
